# baseline (speedup 1.0000x reference)
import functools

import jax
import jax.numpy as jnp
from jax import lax
from jax.experimental import pallas as pl
from jax.experimental.pallas import tpu as pltpu
from jax.experimental.pallas import tpu_sc as plsc


_CHUNK = 16


def _sc_gather(table, flat_idx):
  n = flat_idx.shape[0]
  d = table.shape[1]
  mesh = plsc.VectorSubcoreMesh(core_axis_name="core", subcore_axis_name="subcore")
  nw = mesh.num_cores * mesh.num_subcores
  per_w = n // nw
  nchunk = per_w // _CHUNK

  @functools.partial(
      pl.kernel,
      out_type=jax.ShapeDtypeStruct((n, d), table.dtype),
      mesh=mesh,
      scratch_types=[
          pltpu.VMEM((per_w,), jnp.int32),
          pltpu.VMEM((_CHUNK, d), table.dtype),
          pltpu.VMEM((_CHUNK, d), table.dtype),
          pltpu.SemaphoreType.DMA,
          pltpu.SemaphoreType.DMA,
          pltpu.SemaphoreType.DMA,
          pltpu.SemaphoreType.DMA,
      ],
  )
  def gather_kernel(x_hbm, i_hbm, o_hbm, idx_v, rows0, rows1, g0, g1, s0, s1):
    wid = lax.axis_index("subcore") * mesh.num_cores + lax.axis_index("core")
    base = wid * per_w
    pltpu.sync_copy(i_hbm.at[pl.ds(base, per_w)], idx_v)

    bufs = (rows0, rows1)
    gsem = (g0, g1)
    ssem = (s0, s1)

    def start_gather(j, b):
      return pltpu.async_copy(
          x_hbm.at[idx_v.at[pl.ds(j * _CHUNK, _CHUNK)]], bufs[b], gsem[b]
      )

    def start_store(j, b):
      return pltpu.async_copy(
          bufs[b], o_hbm.at[pl.ds(base + j * _CHUNK, _CHUNK)], ssem[b]
      )

    g_h = [start_gather(0, 0), None]
    s_h = [None, None]
    for j in range(nchunk):
      b = j % 2
      if j + 1 < nchunk:
        if s_h[1 - b] is not None:
          s_h[1 - b].wait()
        g_h[1 - b] = start_gather(j + 1, 1 - b)
      g_h[b].wait()
      s_h[b] = start_store(j, b)
    for h in s_h:
      if h is not None:
        h.wait()

  return gather_kernel(table, flat_idx)


def _mm_body(xg_ref, w_ref, o_ref):
  a = xg_ref[0, 0].astype(jnp.bfloat16)
  b = w_ref[0]
  o_ref[0, 0] = jnp.dot(a, b, preferred_element_type=jnp.float32)


def _tc_matmul(xg, W):
  B, E, K, I = xg.shape
  J = W.shape[2]
  return pl.pallas_call(
      _mm_body,
      grid=(E, B),
      in_specs=[
          pl.BlockSpec((1, 1, K, I), lambda e, b: (b, e, 0, 0)),
          pl.BlockSpec((1, I, J), lambda e, b: (e, 0, 0)),
      ],
      out_specs=pl.BlockSpec((1, 1, K, J), lambda e, b: (b, e, 0, 0)),
      out_shape=jax.ShapeDtypeStruct((B, E, K, J), jnp.float32),
      compiler_params=pltpu.CompilerParams(
          dimension_semantics=("arbitrary", "arbitrary"),
      ),
  )(xg, W)


def kernel(x, Ind, W):
  B, T, I = x.shape
  E, K = Ind.shape[1], Ind.shape[2]
  table = x.reshape(B * T, I)
  flat_idx = (
      jnp.arange(B, dtype=jnp.int32)[:, None, None] * T + Ind
  ).reshape(B * E * K)
  Wb = W.astype(jnp.bfloat16)
  xg = _sc_gather(table, flat_idx).reshape(B, E, K, I)
  return _tc_matmul(xg, Wb)


# speedup vs baseline: 1.1492x; 1.1492x over previous
import functools

import jax
import jax.numpy as jnp
from jax import lax
from jax.experimental import pallas as pl
from jax.experimental.pallas import tpu as pltpu
from jax.experimental.pallas import tpu_sc as plsc


_CHUNK = 16


def _sc_gather(table, flat_idx, offset, nrows):
  d = table.shape[1]
  mesh = plsc.VectorSubcoreMesh(core_axis_name="core", subcore_axis_name="subcore")
  nw = mesh.num_cores * mesh.num_subcores
  per_w = nrows // nw
  nchunk = per_w // _CHUNK

  @functools.partial(
      pl.kernel,
      out_type=jax.ShapeDtypeStruct((nrows, d), table.dtype),
      mesh=mesh,
      scratch_types=[
          pltpu.VMEM((per_w,), jnp.int32),
          pltpu.VMEM((_CHUNK, d), table.dtype),
          pltpu.VMEM((_CHUNK, d), table.dtype),
          pltpu.SemaphoreType.DMA,
          pltpu.SemaphoreType.DMA,
          pltpu.SemaphoreType.DMA,
          pltpu.SemaphoreType.DMA,
      ],
  )
  def gather_kernel(x_hbm, i_hbm, o_hbm, idx_v, rows0, rows1, g0, g1, s0, s1):
    wid = lax.axis_index("subcore") * mesh.num_cores + lax.axis_index("core")
    base = wid * per_w
    pltpu.sync_copy(i_hbm.at[pl.ds(offset + base, per_w)], idx_v)

    bufs = (rows0, rows1)
    gsem = (g0, g1)
    ssem = (s0, s1)

    def start_gather(j, b):
      return pltpu.async_copy(
          x_hbm.at[idx_v.at[pl.ds(j * _CHUNK, _CHUNK)]], bufs[b], gsem[b]
      )

    def start_store(j, b):
      return pltpu.async_copy(
          bufs[b], o_hbm.at[pl.ds(base + j * _CHUNK, _CHUNK)], ssem[b]
      )

    g_h = [start_gather(0, 0), None]
    s_h = [None, None]
    for j in range(nchunk):
      b = j % 2
      if j + 1 < nchunk:
        if s_h[1 - b] is not None:
          s_h[1 - b].wait()
        g_h[1 - b] = start_gather(j + 1, 1 - b)
      g_h[b].wait()
      s_h[b] = start_store(j, b)
    for h in s_h:
      if h is not None:
        h.wait()

  return gather_kernel(table, flat_idx)


def _mm_body(y_in_ref, xg_ref, w_ref, o_ref):
  del y_in_ref
  a = xg_ref[0, 0].astype(jnp.bfloat16)
  b = w_ref[0]
  o_ref[0, 0] = jnp.dot(a.astype(jnp.bfloat16), b.astype(jnp.bfloat16),
                        preferred_element_type=jnp.float32)


def _tc_matmul_chunk(y_prev, xg, W, e0, epc, alias):
  _, B, K, I = xg.shape
  E, J = W.shape[0], W.shape[2]
  kwargs = {}
  if alias:
    kwargs["input_output_aliases"] = {0: 0}
  return pl.pallas_call(
      _mm_body,
      grid=(epc, B),
      in_specs=[
          pl.BlockSpec(memory_space=pl.ANY),
          pl.BlockSpec((1, 1, K, I), lambda e, b: (e, b, 0, 0)),
          pl.BlockSpec((1, I, J), lambda e, b: (e0 + e, 0, 0)),
      ],
      out_specs=pl.BlockSpec((1, 1, K, J), lambda e, b: (b, e0 + e, 0, 0)),
      out_shape=jax.ShapeDtypeStruct((B, E, K, J), jnp.float32),
      compiler_params=pltpu.CompilerParams(
          dimension_semantics=("arbitrary", "arbitrary"),
      ),
      **kwargs,
  )(y_prev, xg, W)


_EPC = 2


def kernel(x, Ind, W):
  B, T, I = x.shape
  E, K = Ind.shape[1], Ind.shape[2]
  J = W.shape[2]
  table = x.reshape(B * T, I)
  flat_idx = (
      jnp.arange(B, dtype=jnp.int32)[None, :, None] * T
      + jnp.transpose(Ind, (1, 0, 2))
  ).reshape(E * B * K)

  nch = E // _EPC
  rows_per_chunk = _EPC * B * K
  xgs = [
      _sc_gather(table, flat_idx, c * rows_per_chunk, rows_per_chunk)
      for c in range(nch)
  ]
  del J
  y = None
  for c in range(nch):
    xg = xgs[c].reshape(_EPC, B, K, I)
    if y is None:
      dummy = jnp.zeros((8, 128), jnp.float32)
      y = _tc_matmul_chunk(dummy, xg, W, 0, _EPC, alias=False)
    else:
      y = _tc_matmul_chunk(y, xg, W, c * _EPC, _EPC, alias=True)
  return y
